# baseline (speedup 1.0000x reference)
import functools

import jax
import jax.numpy as jnp
from jax import lax
from jax.experimental import pallas as pl
from jax.experimental.pallas import tpu as pltpu
from jax.experimental.pallas import tpu_sc as plsc

PHONEME_SIZE = 1000
D = 64
BATCH = 16384
SEQ = 50

_INFO = plsc.get_sparse_core_info()
_NC = _INFO.num_cores
_NS = _INFO.num_subcores
_NW = _NC * _NS
_BT = 128
_NBT = BATCH // _BT
_K = _NBT // _NW
_NJ = _K * SEQ


@functools.partial(
    pl.kernel,
    out_type=jax.ShapeDtypeStruct((SEQ, D // 8, _NBT, 8 * _BT), jnp.float32),
    mesh=plsc.VectorSubcoreMesh(core_axis_name="c", subcore_axis_name="s"),
    compiler_params=pltpu.CompilerParams(
        use_tc_tiling_on_sc=False, needs_layout_passes=False
    ),
    scratch_types=[
        pltpu.VMEM((_K, SEQ, _BT), jnp.int32),
        pltpu.VMEM((_BT, D), jnp.float32),
        pltpu.VMEM((_BT, D), jnp.float32),
        pltpu.VMEM((D * _BT,), jnp.float32),
        pltpu.VMEM((D * _BT,), jnp.float32),
    ]
    + [pltpu.SemaphoreType.DMA] * 4,
)
def _embed_sc(xt_hbm, table_hbm, out_hbm, idx_v, g0_v, g1_v, t0_v, t1_v, *sems):
    g_v = (g0_v, g1_v)
    t_v = (t0_v, t1_v)
    sem_g = sems[:2]
    sem_s = sems[2:]
    wid = lax.axis_index("s") * _NC + lax.axis_index("c")
    bt0 = wid * _K

    for k in range(_K):
        pltpu.sync_copy(xt_hbm.at[:, pl.ds((bt0 + k) * _BT, _BT)], idx_v.at[k])

    def split(j):
        return j & (_K - 1), j >> 2

    def gather(j, b):
        k, s = split(j)
        return pltpu.make_async_copy(
            table_hbm.at[idx_v.at[k, s]], g_v[b], sem_g[b]
        )

    def writeback(j, b, dt):
        k, s = split(j)
        return pltpu.make_async_copy(
            t_v[b].at[pl.ds(dt * 8 * _BT, 8 * _BT)],
            out_hbm.at[s, dt, bt0 + k],
            sem_s[b],
        )

    def wb_start(j, b):
        for dt in range(D // 8):
            writeback(j, b, dt).start()

    def wb_wait(j, b):
        for dt in range(D // 8):
            writeback(j, b, dt).wait()

    colvec = [(c * 16 + lax.iota(jnp.int32, 16)) * _BT for c in range(D // 16)]

    def transpose(b):
        for c in range(D // 16):
            for bi in range(_BT):
                v = g_v[b][bi, pl.ds(c * 16, 16)]
                plsc.store_scatter(t_v[b], [colvec[c] + bi], v)

    gather(0, 0).start()
    gather(1, 1).start()

    def body(t, carry):
        for i in range(2):
            j = 2 * t + i
            gather(j, i).wait()

            @pl.when(t > 0)
            def _():
                wb_wait(j - 2, i)

            transpose(i)
            wb_start(j, i)

            @pl.when(t < _NJ // 2 - 1)
            def _():
                gather(j + 2, i).start()

        return carry

    lax.fori_loop(0, _NJ // 2, body, 0)
    wb_wait(_NJ - 2, 0)
    wb_wait(_NJ - 1, 1)


def kernel(x, weight):
    xt = x.astype(jnp.int32).T
    p = _embed_sc(xt, weight)
    p5 = p.reshape(SEQ, D // 8, _NBT, 8, _BT)
    return p5.transpose(2, 4, 0, 1, 3).reshape(BATCH, SEQ, D)


# speedup vs baseline: 1.8132x; 1.8132x over previous
import functools

import jax
import jax.numpy as jnp
from jax import lax
from jax.experimental import pallas as pl
from jax.experimental.pallas import tpu as pltpu
from jax.experimental.pallas import tpu_sc as plsc

PHONEME_SIZE = 1000
D = 64
BATCH = 16384
SEQ = 50

_INFO = plsc.get_sparse_core_info()
_NC = _INFO.num_cores
_NS = _INFO.num_subcores
_NW = _NC * _NS
_BT = 128
_NBT = BATCH // _BT
_K = _NBT // _NW
_NJ = _K * SEQ


@functools.partial(
    pl.kernel,
    out_type=jax.ShapeDtypeStruct((SEQ, D // 8, _NBT, 8, _BT), jnp.float32),
    mesh=plsc.VectorSubcoreMesh(core_axis_name="c", subcore_axis_name="s"),
    compiler_params=pltpu.CompilerParams(
        use_tc_tiling_on_sc=False, needs_layout_passes=False
    ),
    scratch_types=[
        pltpu.VMEM((_K, SEQ, _BT), jnp.int32),
        pltpu.VMEM((_BT, D), jnp.float32),
        pltpu.VMEM((_BT, D), jnp.float32),
        pltpu.VMEM((D, _BT + 1), jnp.float32),
        pltpu.VMEM((D, _BT + 1), jnp.float32),
    ]
    + [pltpu.SemaphoreType.DMA] * 4,
)
def _embed_sc(xt_hbm, table_hbm, out_hbm, idx_v, g0_v, g1_v, t0_v, t1_v, *sems):
    g_v = (g0_v, g1_v)
    t_v = (t0_v, t1_v)
    sem_g = sems[:2]
    sem_s = sems[2:]
    wid = lax.axis_index("s") * _NC + lax.axis_index("c")
    bt0 = wid * _K

    for k in range(_K):
        pltpu.sync_copy(xt_hbm.at[:, pl.ds((bt0 + k) * _BT, _BT)], idx_v.at[k])

    def split(j):
        return j & (_K - 1), j >> 2

    def gather(j, b):
        k, s = split(j)
        return pltpu.make_async_copy(
            table_hbm.at[idx_v.at[k, s]], g_v[b], sem_g[b]
        )

    def writeback(j, b, dt):
        k, s = split(j)
        return pltpu.make_async_copy(
            t_v[b].at[pl.ds(dt * 8, 8), pl.ds(0, _BT)],
            out_hbm.at[s, dt, bt0 + k],
            sem_s[b],
        )

    def wb_start(j, b):
        for dt in range(D // 8):
            writeback(j, b, dt).start()

    def wb_wait(j, b):
        for dt in range(D // 8):
            writeback(j, b, dt).wait()

    rowvec = [c * 16 + lax.iota(jnp.int32, 16) for c in range(D // 16)]

    def transpose(b):
        for bi in range(_BT):
            col = jnp.full((16,), bi, jnp.int32)
            for c in range(D // 16):
                v = g_v[b][bi, pl.ds(c * 16, 16)]
                plsc.store_scatter(t_v[b], [rowvec[c], col], v)

    gather(0, 0).start()
    gather(1, 1).start()

    def body(t, carry):
        for i in range(2):
            j = 2 * t + i
            gather(j, i).wait()

            @pl.when(t > 0)
            def _():
                wb_wait(j - 2, i)

            transpose(i)
            wb_start(j, i)

            @pl.when(t < _NJ // 2 - 1)
            def _():
                gather(j + 2, i).start()

        return carry

    lax.fori_loop(0, _NJ // 2, body, 0)
    wb_wait(_NJ - 2, 0)
    wb_wait(_NJ - 1, 1)


def kernel(x, weight):
    xt = x.astype(jnp.int32).T
    p = _embed_sc(xt, weight)
    p5 = p.reshape(SEQ, D // 8, _NBT, 8, _BT)
    return p5.transpose(2, 4, 0, 1, 3).reshape(BATCH, SEQ, D)


# speedup vs baseline: 1.8381x; 1.0137x over previous
import functools

import jax
import jax.numpy as jnp
from jax import lax
from jax.experimental import pallas as pl
from jax.experimental.pallas import tpu as pltpu
from jax.experimental.pallas import tpu_sc as plsc

PHONEME_SIZE = 1000
D = 64
BATCH = 16384
SEQ = 50

_INFO = plsc.get_sparse_core_info()
_NC = _INFO.num_cores
_NS = _INFO.num_subcores
_NW = _NC * _NS
_BT = 128
_NBT = BATCH // _BT
_K = _NBT // _NW
_NJ = _K * SEQ
_NBUF = 4


@functools.partial(
    pl.kernel,
    out_type=jax.ShapeDtypeStruct((SEQ, D // 8, _NBT, 8, _BT), jnp.float32),
    mesh=plsc.VectorSubcoreMesh(core_axis_name="c", subcore_axis_name="s"),
    compiler_params=pltpu.CompilerParams(
        use_tc_tiling_on_sc=False, needs_layout_passes=False
    ),
    scratch_types=[pltpu.VMEM((_K, SEQ, _BT), jnp.int32)]
    + [pltpu.VMEM((_BT, D), jnp.float32)] * _NBUF
    + [pltpu.VMEM((D // 8, 8, _BT + 1), jnp.float32)] * _NBUF
    + [pltpu.SemaphoreType.DMA] * (2 * _NBUF),
)
def _embed_sc(xt_hbm, table_hbm, out_hbm, idx_v, *bufs):
    g_v = bufs[:_NBUF]
    t_v = bufs[_NBUF : 2 * _NBUF]
    sem_g = bufs[2 * _NBUF : 3 * _NBUF]
    sem_s = bufs[3 * _NBUF :]
    wid = lax.axis_index("s") * _NC + lax.axis_index("c")
    bt0 = wid * _K

    for k in range(_K):
        pltpu.sync_copy(xt_hbm.at[:, pl.ds((bt0 + k) * _BT, _BT)], idx_v.at[k])

    def split(j):
        return j & (_K - 1), j >> 2

    def gather(j, b):
        k, s = split(j)
        return pltpu.make_async_copy(
            table_hbm.at[idx_v.at[k, s]], g_v[b], sem_g[b]
        )

    def writeback(j, b):
        k, s = split(j)
        return pltpu.make_async_copy(
            t_v[b].at[:, :, pl.ds(0, _BT)], out_hbm.at[s, :, bt0 + k], sem_s[b]
        )

    dtvec = [(c * 16 + lax.iota(jnp.int32, 16)) >> 3 for c in range(D // 16)]
    divec = [(c * 16 + lax.iota(jnp.int32, 16)) & 7 for c in range(D // 16)]

    def transpose(b):
        for bi in range(_BT):
            col = jnp.full((16,), bi, jnp.int32)
            for c in range(D // 16):
                v = g_v[b][bi, pl.ds(c * 16, 16)]
                plsc.store_scatter(t_v[b], [dtvec[c], divec[c], col], v)

    for b in range(_NBUF):
        gather(b, b).start()

    def body(t, carry):
        for i in range(_NBUF):
            j = _NBUF * t + i
            gather(j, i).wait()

            @pl.when(t > 0)
            def _():
                writeback(j - _NBUF, i).wait()

            transpose(i)
            writeback(j, i).start()

            @pl.when(t < _NJ // _NBUF - 1)
            def _():
                gather(j + _NBUF, i).start()

        return carry

    lax.fori_loop(0, _NJ // _NBUF, body, 0)
    for b in range(_NBUF):
        writeback(_NJ - _NBUF + b, b).wait()


def kernel(x, weight):
    xt = x.astype(jnp.int32).T
    p = _embed_sc(xt, weight)
    return p.transpose(2, 4, 0, 1, 3).reshape(BATCH, SEQ, D)


# speedup vs baseline: 1.9553x; 1.0638x over previous
import functools

import jax
import jax.numpy as jnp
from jax import lax
from jax.experimental import pallas as pl
from jax.experimental.pallas import tpu as pltpu
from jax.experimental.pallas import tpu_sc as plsc

PHONEME_SIZE = 1000
D = 64
_DP = D + 1
BATCH = 16384
SEQ = 50

_INFO = plsc.get_sparse_core_info()
_NC = _INFO.num_cores
_NS = _INFO.num_subcores
_NW = _NC * _NS
_BT = 128
_NBT = BATCH // _BT
_K = _NBT // _NW
_NJ = _K * SEQ
_NBUF = 4


@functools.partial(
    pl.kernel,
    out_type=jax.ShapeDtypeStruct((SEQ, D // 8, _NBT, 8, _BT), jnp.float32),
    mesh=plsc.VectorSubcoreMesh(core_axis_name="c", subcore_axis_name="s"),
    compiler_params=pltpu.CompilerParams(
        use_tc_tiling_on_sc=False, needs_layout_passes=False
    ),
    scratch_types=[pltpu.VMEM((_K, SEQ, _BT), jnp.int32)]
    + [pltpu.VMEM((_BT, _DP), jnp.float32)] * _NBUF
    + [pltpu.VMEM((D // 8, 8, _BT), jnp.float32)] * _NBUF
    + [pltpu.SemaphoreType.DMA] * (2 * _NBUF),
)
def _embed_sc(xt_hbm, table_hbm, out_hbm, idx_v, *bufs):
    g_v = bufs[:_NBUF]
    t_v = bufs[_NBUF : 2 * _NBUF]
    sem_g = bufs[2 * _NBUF : 3 * _NBUF]
    sem_s = bufs[3 * _NBUF :]
    wid = lax.axis_index("s") * _NC + lax.axis_index("c")
    bt0 = wid * _K

    for k in range(_K):
        pltpu.sync_copy(xt_hbm.at[:, pl.ds((bt0 + k) * _BT, _BT)], idx_v.at[k])

    def split(j):
        return j & (_K - 1), j >> 2

    def gather(j, b):
        k, s = split(j)
        return pltpu.make_async_copy(
            table_hbm.at[idx_v.at[k, s]], g_v[b], sem_g[b]
        )

    def writeback(j, b):
        k, s = split(j)
        return pltpu.make_async_copy(t_v[b], out_hbm.at[s, :, bt0 + k], sem_s[b])

    rows = [g * 16 + lax.iota(jnp.int32, 16) for g in range(_BT // 16)]

    def transpose(b):
        for dt in range(D // 8):
            for di in range(8):
                col = jnp.full((16,), dt * 8 + di, jnp.int32)
                vs = [
                    plsc.load_gather(g_v[b], [rows[g], col])
                    for g in range(_BT // 16)
                ]
                for g in range(_BT // 16):
                    t_v[b][dt, di, pl.ds(g * 16, 16)] = vs[g]

    for b in range(_NBUF):
        gather(b, b).start()

    def body(t, carry):
        for i in range(_NBUF):
            j = _NBUF * t + i
            gather(j, i).wait()

            @pl.when(t > 0)
            def _():
                writeback(j - _NBUF, i).wait()

            transpose(i)
            writeback(j, i).start()

            @pl.when(t < _NJ // _NBUF - 1)
            def _():
                gather(j + _NBUF, i).start()

        return carry

    lax.fori_loop(0, _NJ // _NBUF, body, 0)
    for b in range(_NBUF):
        writeback(_NJ - _NBUF + b, b).wait()


def kernel(x, weight):
    xt = x.astype(jnp.int32).T
    wp = jnp.pad(weight, ((0, 0), (0, _DP - D)))
    p = _embed_sc(xt, wp)
    return p.transpose(2, 4, 0, 1, 3).reshape(BATCH, SEQ, D)
